# baseline (speedup 1.0000x reference)
import jax
import jax.numpy as jnp
from jax.experimental import pallas as pl

_SEQ = 2048
_UNITS = 32
_BI = 64
_BJ = 256


def _tc_body(p_ref, out_ref):
    i0 = pl.program_id(0) * _BI
    j0 = pl.program_id(1) * _BJ

    all_p0 = j0 + _BJ <= i0 - 1
    all_p4 = j0 >= i0 + _BI + 1

    @pl.when(all_p0)
    def _():
        out_ref[...] = jnp.broadcast_to(p_ref[0][None], (_BI, _UNITS, _BJ))

    @pl.when(all_p4)
    def _():
        out_ref[...] = jnp.broadcast_to(p_ref[4][None], (_BI, _UNITS, _BJ))

    @pl.when(jnp.logical_not(jnp.logical_or(all_p0, all_p4)))
    def _():
        i = i0 + jax.lax.broadcasted_iota(jnp.int32, (_BI, 1, _BJ), 0)
        j = j0 + jax.lax.broadcasted_iota(jnp.int32, (_BI, 1, _BJ), 2)
        d = j - i
        p = [p_ref[v][None] for v in range(5)]
        out_ref[...] = jnp.where(
            d <= -2, p[0],
            jnp.where(d == -1, p[1],
                      jnp.where(d == 0, p[2],
                                jnp.where(d == 1, p[3], p[4]))))


def kernel(embedding):
    planes = jnp.broadcast_to(embedding[:, :, None], (5, _UNITS, _BJ))
    out_t = pl.pallas_call(
        _tc_body,
        grid=(_SEQ // _BI, _SEQ // _BJ),
        in_specs=[pl.BlockSpec((5, _UNITS, _BJ), lambda i, j: (0, 0, 0))],
        out_specs=pl.BlockSpec((_BI, _UNITS, _BJ), lambda i, j: (i, 0, j)),
        out_shape=jax.ShapeDtypeStruct((_SEQ, _UNITS, _SEQ), jnp.float32),
    )(planes)
    return out_t.transpose(0, 2, 1)


# speedup vs baseline: 1.2552x; 1.2552x over previous
import jax
import jax.numpy as jnp
from jax.experimental import pallas as pl

_SEQ = 2048
_UNITS = 32
_BI = 64
_BC = 128


def _tc_body(p_ref, out_ref):
    i0 = pl.program_id(0) * _BI
    c_lo = (i0 - 1) // _BC
    c_hi = (i0 + _BI) // _BC

    for c in range(_SEQ // _BC):
        sub = pl.ds(c * _BC, _BC)
        diag = jnp.logical_or(c == c_lo, c == c_hi)

        @pl.when(diag)
        def _(c=c, sub=sub):
            i = i0 + jax.lax.broadcasted_iota(jnp.int32, (_BI, 1, _BC), 0)
            j = c * _BC + jax.lax.broadcasted_iota(jnp.int32, (_BI, 1, _BC), 2)
            d = j - i
            p = [p_ref[v, :, sub][None] for v in range(5)]
            out_ref[:, :, sub] = jnp.where(
                d <= -2, p[0],
                jnp.where(d == -1, p[1],
                          jnp.where(d == 0, p[2],
                                    jnp.where(d == 1, p[3], p[4]))))

        @pl.when(jnp.logical_not(diag))
        def _(c=c, sub=sub):
            val = jnp.where(c * _BC < i0, p_ref[0, :, sub], p_ref[4, :, sub])
            out_ref[:, :, sub] = jnp.broadcast_to(val[None], (_BI, _UNITS, _BC))


def kernel(embedding):
    planes = jnp.broadcast_to(embedding[:, :, None], (5, _UNITS, _SEQ))
    out_t = pl.pallas_call(
        _tc_body,
        grid=(_SEQ // _BI,),
        in_specs=[pl.BlockSpec((5, _UNITS, _SEQ), lambda i: (0, 0, 0))],
        out_specs=pl.BlockSpec((_BI, _UNITS, _SEQ), lambda i: (i, 0, 0)),
        out_shape=jax.ShapeDtypeStruct((_SEQ, _UNITS, _SEQ), jnp.float32),
    )(planes)
    return out_t.transpose(0, 2, 1)


# speedup vs baseline: 1.3049x; 1.0396x over previous
import jax
import jax.numpy as jnp
from jax.experimental import pallas as pl

_SEQ = 2048
_UNITS = 32
_BI = 64
_BC = 256


def _tc_body(p_ref, out_ref):
    i0 = pl.program_id(0) * _BI
    c_lo = (i0 - 1) // _BC
    c_hi = (i0 + _BI) // _BC

    for c in range(_SEQ // _BC):
        sub = pl.ds(c * _BC, _BC)
        diag = jnp.logical_or(c == c_lo, c == c_hi)

        @pl.when(diag)
        def _(c=c, sub=sub):
            i = i0 + jax.lax.broadcasted_iota(jnp.int32, (_BI, 1, _BC), 0)
            j = c * _BC + jax.lax.broadcasted_iota(jnp.int32, (_BI, 1, _BC), 2)
            d = j - i
            p = [p_ref[v, :, sub][None] for v in range(5)]
            out_ref[:, :, sub] = jnp.where(
                d <= -2, p[0],
                jnp.where(d == -1, p[1],
                          jnp.where(d == 0, p[2],
                                    jnp.where(d == 1, p[3], p[4]))))

        @pl.when(jnp.logical_not(diag))
        def _(c=c, sub=sub):
            val = jnp.where(c * _BC < i0, p_ref[0, :, sub], p_ref[4, :, sub])
            out_ref[:, :, sub] = jnp.broadcast_to(val[None], (_BI, _UNITS, _BC))


def kernel(embedding):
    planes = jnp.broadcast_to(embedding[:, :, None], (5, _UNITS, _SEQ))
    out_t = pl.pallas_call(
        _tc_body,
        grid=(_SEQ // _BI,),
        in_specs=[pl.BlockSpec((5, _UNITS, _SEQ), lambda i: (0, 0, 0))],
        out_specs=pl.BlockSpec((_BI, _UNITS, _SEQ), lambda i: (i, 0, 0)),
        out_shape=jax.ShapeDtypeStruct((_SEQ, _UNITS, _SEQ), jnp.float32),
    )(planes)
    return out_t.transpose(0, 2, 1)


# speedup vs baseline: 1.3131x; 1.0063x over previous
import jax
import jax.numpy as jnp
from jax.experimental import pallas as pl

_SEQ = 2048
_UNITS = 32
_BI = 64
_BC = 512


def _tc_body(p_ref, out_ref):
    i0 = pl.program_id(0) * _BI
    c_lo = (i0 - 1) // _BC
    c_hi = (i0 + _BI) // _BC

    for c in range(_SEQ // _BC):
        sub = pl.ds(c * _BC, _BC)
        diag = jnp.logical_or(c == c_lo, c == c_hi)

        @pl.when(diag)
        def _(c=c, sub=sub):
            i = i0 + jax.lax.broadcasted_iota(jnp.int32, (_BI, 1, _BC), 0)
            j = c * _BC + jax.lax.broadcasted_iota(jnp.int32, (_BI, 1, _BC), 2)
            d = j - i
            p = [p_ref[v, :, sub][None] for v in range(5)]
            out_ref[:, :, sub] = jnp.where(
                d <= -2, p[0],
                jnp.where(d == -1, p[1],
                          jnp.where(d == 0, p[2],
                                    jnp.where(d == 1, p[3], p[4]))))

        @pl.when(jnp.logical_not(diag))
        def _(c=c, sub=sub):
            val = jnp.where(c * _BC < i0, p_ref[0, :, sub], p_ref[4, :, sub])
            out_ref[:, :, sub] = jnp.broadcast_to(val[None], (_BI, _UNITS, _BC))


def kernel(embedding):
    planes = jnp.broadcast_to(embedding[:, :, None], (5, _UNITS, _SEQ))
    out_t = pl.pallas_call(
        _tc_body,
        grid=(_SEQ // _BI,),
        in_specs=[pl.BlockSpec((5, _UNITS, _SEQ), lambda i: (0, 0, 0))],
        out_specs=pl.BlockSpec((_BI, _UNITS, _SEQ), lambda i: (i, 0, 0)),
        out_shape=jax.ShapeDtypeStruct((_SEQ, _UNITS, _SEQ), jnp.float32),
    )(planes)
    return out_t.transpose(0, 2, 1)
